# baseline (speedup 1.0000x reference)
import functools

import jax
import jax.numpy as jnp
from jax import lax
from jax.experimental import pallas as pl
from jax.experimental.pallas import tpu as pltpu
from jax.experimental.pallas import tpu_sc as plsc

_DIM = 32
_BATCH = 16384
_NELEM = 2 * _BATCH
_NENT = 1000000
_NW = 32
_PAIRS_PER_W = _BATCH // _NW
_ROWS_PER_W = 2 * _PAIRS_PER_W
_RPW = 31232
_WIN = 768
_NWIN = 42
_WIN_CLAMP = 999168
_TAIL_LO = 999936
_TAIL_W = 64
_KEEP_CAP = 1280
_PACK = _KEEP_CAP // 4
_WCAP = 256
_WGROUPS = 4
_MAPCH = _KEEP_CAP // 128
_TRASH = _NELEM
_MAP_PAD = 16
_NEWTON_ITERS = 6
_EXP_EXT = True


def _extract_body(idx_hbm, table_hbm, val_hbm, map_hbm,
                  idx_v, kept_r, kept_e, wr, we, buf, buft, extbuf,
                  eidg, kposb, semw0, semw1, sem_s):
    wid = lax.axis_index("s") * 2 + lax.axis_index("c")
    lo = wid * _RPW
    hi = jnp.where(wid == _NW - 1, _NENT, lo + _RPW)
    lanes = jnp.arange(16, dtype=jnp.int32)

    for j in range(_MAPCH):
        for k in range(8):
            eidg[j, pl.ds(k * 16, 16)] = jnp.full((16,), _TRASH, jnp.int32)
            kposb[j, pl.ds(k * 16, 16)] = (
                wid * _KEEP_CAP + j * 128 + k * 16
            ) + lanes

    cnt_vec = jnp.zeros((16,), jnp.int32)
    for half in range(2):
        pltpu.sync_copy(idx_hbm.at[pl.ds(half * 128, 128)], idx_v)

        def scan_row(j, cnt_vec, half=half):
            for k in range(8):
                rv = idx_v[j, pl.ds(k * 16, 16)]
                m = (rv >= lo) & (rv < hi)
                eid = (half * 16384 + j * 128 + k * 16) + lanes
                pos = cnt_vec + plsc.cumsum(m.astype(jnp.int32)) - 1
                plsc.store_scatter(kept_r, [pos], rv, mask=m)
                plsc.store_scatter(kept_e, [pos], eid, mask=m)
                cnt_vec = cnt_vec + plsc.all_reduce_population_count(m)
            return cnt_vec
        cnt_vec = lax.fori_loop(0, 128, scan_row, cnt_vec)
    nvec = lax.shift_right_logical(cnt_vec[0] + 15, 4)

    sems = (semw0, semw1)

    def wcompact_win(win_lo, win_hi, kbase_vec):
        for i in range(_WGROUPS + 1):
            wr[pl.ds(i * 16, 16)] = jnp.full((16,), 0, jnp.int32) + win_lo
            we[pl.ds(i * 16, 16)] = jnp.full((16,), _TRASH, jnp.int32)

        def wcompact(i, wcnt_vec):
            rv = kept_r[pl.ds(i * 16, 16)]
            ev = kept_e[pl.ds(i * 16, 16)]
            m = (rv >= win_lo) & (rv < win_hi)
            pos = wcnt_vec + plsc.cumsum(m.astype(jnp.int32)) - 1
            plsc.store_scatter(wr, [pos], rv, mask=m)
            plsc.store_scatter(we, [pos], ev, mask=m)
            return wcnt_vec + plsc.all_reduce_population_count(m)
        return lax.fori_loop(0, nvec, wcompact, jnp.zeros((16,), jnp.int32))

    def ext_win(bufk, win_lo, kbase_vec, ng):
        for grp in range(_WGROUPS):
            @pl.when(grp < ng)
            def _ext(grp=grp):
                rel = wr[pl.ds(grp * 16, 16)] - win_lo
                ev = we[pl.ds(grp * 16, 16)]
                kpos = (kbase_vec + grp * 16) + lanes
                prow = lax.shift_right_logical(kpos, 2)
                pcol = (kpos & 3) * _DIM
                for d in range(_DIM):
                    ud = plsc.load_gather(
                        bufk, [jnp.full((16,), d, jnp.int32), rel]
                    )
                    plsc.store_scatter(extbuf, [prow, pcol + d], ud)
                plsc.store_scatter(
                    eidg, [lax.shift_right_logical(kpos, 7), kpos & 127], ev
                )

    def win_lo_of(k):
        return pl.multiple_of(
            jnp.minimum(lo + _WIN * k, _WIN_CLAMP), 128
        )

    def fire(k, p):
        return pltpu.async_copy(
            table_hbm.at[:, pl.ds(win_lo_of(k), _WIN)], buf.at[p], sems[p]
        )

    fire(0, 0)
    fire(1, 1)

    def win_iter(k, kbase_vec):
        win_lo = win_lo_of(k)
        win_hi = win_lo + _WIN
        wcnt_vec = wcompact_win(win_lo, win_hi, kbase_vec)
        ng = lax.shift_right_logical(wcnt_vec[0] + 15, 4)
        for p in range(2):
            @pl.when((k & 1) == p)
            def _do(p=p):
                pltpu.make_async_copy(
                    table_hbm.at[:, pl.ds(win_lo, _WIN)], buf.at[p], sems[p]
                ).wait()
                ext_win(buf.at[p], win_lo, kbase_vec, ng)

                @pl.when(k + 2 < _NWIN)
                def _fire_next():
                    fire(k + 2, p)
        return kbase_vec + wcnt_vec

    kbase_vec = lax.fori_loop(
        0, _NWIN, win_iter, jnp.zeros((16,), jnp.int32)
    )

    tail_lo = jnp.int32(_TAIL_LO)
    pltpu.async_copy(
        table_hbm.at[:, pl.ds(tail_lo, _TAIL_W)], buft, sems[0]
    ).wait()
    wcnt_vec = wcompact_win(tail_lo, tail_lo + _TAIL_W, kbase_vec)
    ng = lax.shift_right_logical(wcnt_vec[0] + 15, 4)
    ext_win(buft, tail_lo, kbase_vec, ng)

    pltpu.sync_copy(extbuf, val_hbm.at[pl.ds(wid * _PACK, _PACK)])
    for j in range(_MAPCH):
        pltpu.async_copy(kposb.at[j], map_hbm.at[eidg.at[j]], sem_s)
    for j in range(_MAPCH):
        pltpu.make_async_copy(
            map_hbm.at[pl.ds(0, 128)], kposb.at[0], sem_s
        ).wait()


def _pairs_body(val_hbm, map_hbm, out_hbm, map_v, idxb, gbuf, out_v,
                sem0, sem1):
    wid = lax.axis_index("s") * 2 + lax.axis_index("c")
    lanes = jnp.arange(16, dtype=jnp.int32)
    ebase = wid * _ROWS_PER_W
    sems = (sem0, sem1)

    pltpu.sync_copy(map_hbm.at[pl.ds(ebase, _ROWS_PER_W)], map_v)

    for j in range(8):
        for k in range(8):
            gs = map_v[pl.ds(j * 128 + k * 16, 16)]
            idxb[j, pl.ds(k * 16, 16)] = lax.shift_right_logical(gs, 2)

    def fire(c):
        return pltpu.async_copy(
            val_hbm.at[idxb.at[c]], gbuf.at[c % 2], sems[c % 2]
        )

    copies = [None, None]
    copies[0] = fire(0)
    for c in range(8):
        if c + 1 < 8:
            copies[(c + 1) % 2] = fire(c + 1)
        copies[c % 2].wait()
        bufc = gbuf.at[c % 2]

        def group_body(g, carry, bufc=bufc, c=c):
            eoff = 128 * c + 32 * g
            qu = eoff + 2 * lanes
            gs_u = plsc.load_gather(map_v, [qu])
            gs_v = plsc.load_gather(map_v, [qu + 1])
            row_u = (32 * g) + 2 * lanes
            row_v = row_u + 1
            col_u = (gs_u & 3) * _DIM
            col_v = (gs_v & 3) * _DIM
            uu = jnp.zeros((16,), jnp.float32)
            vv = jnp.zeros((16,), jnp.float32)
            uv = jnp.zeros((16,), jnp.float32)
            for d in range(_DIM):
                u = plsc.load_gather(bufc, [row_u, col_u + d])
                v = plsc.load_gather(bufc, [row_v, col_v + d])
                uu = uu + u * u
                vv = vv + v * v
                uv = uv + u * v
            x = (uu + 1.0) * (vv + 1.0)
            y = 0.5 * (x + 1.0)
            for _ in range(_NEWTON_ITERS):
                y = 0.5 * (y + x / y)
            dist = 2.0 * y - 2.0 * uv + (-2.0 + 1e-5)
            out_v[pl.ds(c * 64 + g * 16, 16)] = dist
            return carry

        lax.fori_loop(0, 4, group_body, 0)

    pltpu.sync_copy(out_v, out_hbm.at[pl.ds(wid * _PAIRS_PER_W, _PAIRS_PER_W)])


@functools.partial(jax.jit, static_argnums=())
def kernel(idxs, table):
    idx_flat = idxs.reshape(_NELEM // 128, 128)
    table_t = table.T
    mesh = plsc.VectorSubcoreMesh(core_axis_name="c", subcore_axis_name="s")
    run_a = pl.kernel(
        _extract_body,
        out_type=(
            jax.ShapeDtypeStruct((_NW * _PACK, 128), jnp.float32),
            jax.ShapeDtypeStruct((_NELEM + _MAP_PAD,), jnp.int32),
        ),
        mesh=mesh,
        scratch_types=[
            pltpu.VMEM((128, 128), jnp.int32),
            pltpu.VMEM((_KEEP_CAP + 16,), jnp.int32),
            pltpu.VMEM((_KEEP_CAP + 16,), jnp.int32),
            pltpu.VMEM((_WCAP + 16,), jnp.int32),
            pltpu.VMEM((_WCAP + 16,), jnp.int32),
            pltpu.VMEM((2, _DIM, _WIN), jnp.float32),
            pltpu.VMEM((_DIM, _TAIL_W), jnp.float32),
            pltpu.VMEM((_PACK, 128), jnp.float32),
            pltpu.VMEM((_MAPCH, 128), jnp.int32),
            pltpu.VMEM((_MAPCH, 128), jnp.int32),
            pltpu.SemaphoreType.DMA,
            pltpu.SemaphoreType.DMA,
            pltpu.SemaphoreType.DMA,
        ],
        compiler_params=pltpu.CompilerParams(needs_layout_passes=False),
    )
    val, slot_map = run_a(idx_flat, table_t)
    run_b = pl.kernel(
        _pairs_body,
        out_type=jax.ShapeDtypeStruct((_BATCH,), jnp.float32),
        mesh=mesh,
        scratch_types=[
            pltpu.VMEM((_ROWS_PER_W,), jnp.int32),
            pltpu.VMEM((8, 128), jnp.int32),
            pltpu.VMEM((2, 128, 128), jnp.float32),
            pltpu.VMEM((_PAIRS_PER_W,), jnp.float32),
            pltpu.SemaphoreType.DMA,
            pltpu.SemaphoreType.DMA,
        ],
        compiler_params=pltpu.CompilerParams(needs_layout_passes=False),
    )
    return run_b(val, slot_map)


# speedup vs baseline: 2.6952x; 2.6952x over previous
import functools

import jax
import jax.numpy as jnp
from jax import lax
from jax.experimental import pallas as pl
from jax.experimental.pallas import tpu as pltpu
from jax.experimental.pallas import tpu_sc as plsc

_DIM = 32
_BATCH = 16384
_NW = 32
_PAIRS_PER_W = _BATCH // _NW
_ROWS_PER_W = 2 * _PAIRS_PER_W
_CHUNK = 128
_NCHUNK = _ROWS_PER_W // _CHUNK
_GROUPS = _PAIRS_PER_W // 16
_NEWTON_ITERS = 6


def _sc_body(idx_hbm, table_hbm, out_hbm, idx_v, rows_v, out_v, sem):
    wid = lax.axis_index("s") * 2 + lax.axis_index("c")

    pltpu.sync_copy(idx_hbm.at[pl.ds(wid * _NCHUNK, _NCHUNK)], idx_v)

    copies = [
        pltpu.async_copy(
            table_hbm.at[idx_v.at[j]],
            rows_v.at[pl.ds(j * _CHUNK, _CHUNK)],
            sem,
        )
        for j in range(_NCHUNK)
    ]
    for c in copies:
        c.wait()

    lanes = jnp.arange(16, dtype=jnp.int32)

    def group_body(g, carry):
        row_u = g * 32 + 2 * lanes
        row_v = row_u + 1
        uu = jnp.zeros((16,), jnp.float32)
        vv = jnp.zeros((16,), jnp.float32)
        uv = jnp.zeros((16,), jnp.float32)
        for d in range(_DIM):
            col = jnp.full((16,), d, dtype=jnp.int32)
            u = plsc.load_gather(rows_v, [row_u, col])
            v = plsc.load_gather(rows_v, [row_v, col])
            uu = uu + u * u
            vv = vv + v * v
            uv = uv + u * v
        x = (uu + 1.0) * (vv + 1.0)
        y = 0.5 * (x + 1.0)
        for _ in range(_NEWTON_ITERS):
            y = 0.5 * (y + x / y)
        dist = 2.0 * y - 2.0 * uv + (-2.0 + 1e-5)
        out_v[pl.ds(g * 16, 16)] = dist
        return carry

    lax.fori_loop(0, _GROUPS, group_body, 0)

    pltpu.sync_copy(out_v, out_hbm.at[pl.ds(wid * _PAIRS_PER_W, _PAIRS_PER_W)])


@functools.partial(jax.jit, static_argnums=())
def kernel(idxs, table):
    idx_flat = idxs.reshape(_NW * _NCHUNK, _CHUNK)
    run = pl.kernel(
        _sc_body,
        out_type=jax.ShapeDtypeStruct((_BATCH,), jnp.float32),
        mesh=plsc.VectorSubcoreMesh(core_axis_name="c", subcore_axis_name="s"),
        scratch_types=[
            pltpu.VMEM((_NCHUNK, _CHUNK), jnp.int32),
            pltpu.VMEM((_ROWS_PER_W, _DIM), jnp.float32),
            pltpu.VMEM((_PAIRS_PER_W,), jnp.float32),
            pltpu.SemaphoreType.DMA,
        ],
        compiler_params=pltpu.CompilerParams(
            needs_layout_passes=False, use_tc_tiling_on_sc=False
        ),
    )
    return run(idx_flat, table)


# speedup vs baseline: 2.7334x; 1.0141x over previous
import functools

import jax
import jax.numpy as jnp
from jax import lax
from jax.experimental import pallas as pl
from jax.experimental.pallas import tpu as pltpu
from jax.experimental.pallas import tpu_sc as plsc

_DIM = 32
_BATCH = 16384
_NW = 32
_PAIRS_PER_W = _BATCH // _NW
_ROWS_PER_W = 2 * _PAIRS_PER_W
_CHUNK = 128
_NCHUNK = _ROWS_PER_W // _CHUNK
_GPC = _CHUNK // 32
_NEWTON_ITERS = 6


def _sc_body(idx_hbm, table_hbm, out_hbm, idx_v, rows_v, out_v, sem0, sem1):
    wid = lax.axis_index("s") * 2 + lax.axis_index("c")

    pltpu.sync_copy(idx_hbm.at[pl.ds(wid * _NCHUNK, _NCHUNK)], idx_v)

    sems = (sem0, sem1)

    def fire(j):
        return pltpu.async_copy(
            table_hbm.at[idx_v.at[j]], rows_v.at[j % 2], sems[j % 2]
        )

    lanes = jnp.arange(16, dtype=jnp.int32)

    copies = [None, None]
    copies[0] = fire(0)
    for c in range(_NCHUNK):
        if c + 1 < _NCHUNK:
            copies[(c + 1) % 2] = fire(c + 1)
        copies[c % 2].wait()
        buf = rows_v.at[c % 2]

        def group_body(g, carry, buf=buf, c=c):
            row_u = 32 * g + 2 * lanes
            row_v = row_u + 1
            uu = jnp.zeros((16,), jnp.float32)
            vv = jnp.zeros((16,), jnp.float32)
            uv = jnp.zeros((16,), jnp.float32)
            for d in range(_DIM):
                col = jnp.full((16,), d, dtype=jnp.int32)
                u = plsc.load_gather(buf, [row_u, col])
                v = plsc.load_gather(buf, [row_v, col])
                uu = uu + u * u
                vv = vv + v * v
                uv = uv + u * v
            x = (uu + 1.0) * (vv + 1.0)
            y = 0.5 * (x + 1.0)
            for _ in range(_NEWTON_ITERS):
                y = 0.5 * (y + x / y)
            dist = 2.0 * y - 2.0 * uv + (-2.0 + 1e-5)
            out_v[pl.ds(c * 64 + g * 16, 16)] = dist
            return carry

        lax.fori_loop(0, _GPC, group_body, 0)

    pltpu.sync_copy(out_v, out_hbm.at[pl.ds(wid * _PAIRS_PER_W, _PAIRS_PER_W)])


@functools.partial(jax.jit, static_argnums=())
def kernel(idxs, table):
    idx_flat = idxs.reshape(_NW * _NCHUNK, _CHUNK)
    table_wide = jnp.pad(table, ((0, 0), (0, 128 - _DIM)))
    run = pl.kernel(
        _sc_body,
        out_type=jax.ShapeDtypeStruct((_BATCH,), jnp.float32),
        mesh=plsc.VectorSubcoreMesh(core_axis_name="c", subcore_axis_name="s"),
        scratch_types=[
            pltpu.VMEM((_NCHUNK, _CHUNK), jnp.int32),
            pltpu.VMEM((2, _CHUNK, 128), jnp.float32),
            pltpu.VMEM((_PAIRS_PER_W,), jnp.float32),
            pltpu.SemaphoreType.DMA,
            pltpu.SemaphoreType.DMA,
        ],
        compiler_params=pltpu.CompilerParams(needs_layout_passes=False),
    )
    return run(idx_flat, table_wide)
